# baseline (speedup 1.0000x reference)
import jax
import jax.numpy as jnp
from jax import lax
from jax.experimental import pallas as pl
from jax.experimental.pallas import tpu as pltpu
from jax.experimental.pallas import tpu_sc as plsc

N_FIELDS_K = 26
N_CAT = 100000
B_TOTAL = 16384
NW = 32
B_PER = B_TOTAL // NW
CHUNK = 128
N_BLK = B_PER // CHUNK
TAIL = 64
SPLIT = N_FIELDS_K * N_CAT - TAIL
LAST_LEN = N_CAT - TAIL


def _body(x_hbm, wm_hbm, wt_hbm, bias_hbm, out_hbm,
          idx_v, vals_v, acc_v, idx25_v, tail_v, bias_v, sem):
    c = lax.axis_index("c")
    s = lax.axis_index("s")
    wid = s * 2 + c
    base = wid * B_PER

    pltpu.sync_copy(x_hbm.at[:, pl.ds(base, B_PER)], idx_v)
    pltpu.sync_copy(bias_hbm, bias_v)
    pltpu.sync_copy(wt_hbm, tail_v)

    def save_clamp(i, _):
        st = i * 16
        v = idx_v[25, pl.ds(st, 16)]
        idx25_v[pl.ds(st, 16)] = v
        idx_v[25, pl.ds(st, 16)] = jnp.minimum(v, LAST_LEN - 1)
        return 0

    lax.fori_loop(0, B_PER // 16, save_clamp, 0)

    def fire(f, _):
        pltpu.async_copy(
            wm_hbm.at[0, pl.ds(f * N_CAT, N_CAT)].at[idx_v.at[f]],
            vals_v.at[f], sem)
        return 0

    lax.fori_loop(0, N_FIELDS_K - 1, fire, 0)

    pltpu.async_copy(
        wm_hbm.at[0, pl.ds(25 * N_CAT, LAST_LEN)].at[idx_v.at[25]],
        vals_v.at[25], sem)

    def drain(f, _):
        pltpu.make_async_copy(
            wm_hbm.at[0, pl.ds(0, N_CAT)].at[idx_v.at[0]],
            vals_v.at[f], sem).wait()
        return 0

    lax.fori_loop(0, N_FIELDS_K, drain, 0)

    vbias = bias_v[pl.ds(0, 16)]

    def reduce_chunk(i, _):
        st = i * 16
        acc = vbias
        for f in range(N_FIELDS_K - 1):
            acc = acc + vals_v[f, pl.ds(st, 16)]
        iv = idx25_v[pl.ds(st, 16)]
        in_tail = iv >= LAST_LEN
        tfix = plsc.load_gather(
            tail_v, [jnp.maximum(iv - LAST_LEN, 0)])
        acc = acc + jnp.where(in_tail, tfix, vals_v[25, pl.ds(st, 16)])
        acc_v[pl.ds(st, 16)] = acc
        return 0

    lax.fori_loop(0, B_PER // 16, reduce_chunk, 0)

    pltpu.sync_copy(acc_v, out_hbm.at[pl.ds(base, B_PER)])


@jax.jit
def _cat_linear(x_t, w_main, w_tail, bias16):
    mesh = plsc.VectorSubcoreMesh(core_axis_name="c", subcore_axis_name="s")
    f = pl.kernel(
        _body,
        out_type=jax.ShapeDtypeStruct((B_TOTAL,), jnp.float32),
        mesh=mesh,
        compiler_params=pltpu.CompilerParams(
            needs_layout_passes=False,
            skip_device_barrier=True,
            disable_bounds_checks=True,
            disable_semaphore_checks=True,
            use_tc_tiling_on_sc=False,
        ),
        scratch_types=[
            pltpu.VMEM((N_FIELDS_K, B_PER), jnp.int32),
            pltpu.VMEM((N_FIELDS_K, B_PER), jnp.float32),
            pltpu.VMEM((B_PER,), jnp.float32),
            pltpu.VMEM((B_PER,), jnp.int32),
            pltpu.VMEM((TAIL,), jnp.float32),
            pltpu.VMEM((16,), jnp.float32),
            pltpu.SemaphoreType.DMA,
        ],
    )
    return f(x_t, w_main, w_tail, bias16)


def kernel(x_cat, W, bias):
    x_t = x_cat.T
    w_row = W.T
    w_main = w_row[:, :SPLIT]
    w_tail = W[SPLIT:, 0]
    bias16 = jnp.broadcast_to(bias.reshape(()), (16,)).astype(jnp.float32)
    out = _cat_linear(x_t, w_main, w_tail, bias16)
    return out.reshape(B_TOTAL, 1)


# speedup vs baseline: 1.0316x; 1.0316x over previous
import jax
import jax.numpy as jnp
from jax import lax
from jax.experimental import pallas as pl
from jax.experimental.pallas import tpu as pltpu
from jax.experimental.pallas import tpu_sc as plsc

N_FIELDS_K = 26
N_CAT = 100000
B_TOTAL = 16384
NW = 32
B_PER = B_TOTAL // NW
TAIL = 64
SPLIT = N_FIELDS_K * N_CAT - TAIL
LAST_LEN = N_CAT - TAIL


def _body(x_hbm, wm_hbm, aux_hbm, out_hbm,
          idx_v, vals_v, acc_v, idx25_v, aux_v, sems):
    c = lax.axis_index("c")
    s = lax.axis_index("s")
    wid = s * 2 + c
    base = wid * B_PER

    pltpu.sync_copy(x_hbm.at[:, pl.ds(base, B_PER)], idx_v)
    pltpu.sync_copy(aux_hbm, aux_v)

    def save_clamp(i, _):
        st = i * 16
        v = idx_v[25, pl.ds(st, 16)]
        idx25_v[pl.ds(st, 16)] = v
        idx_v[25, pl.ds(st, 16)] = jnp.minimum(v, LAST_LEN - 1)
        return 0

    lax.fori_loop(0, B_PER // 16, save_clamp, 0)

    def fire(f, _):
        pltpu.async_copy(
            wm_hbm.at[0, pl.ds(f * N_CAT, N_CAT)].at[idx_v.at[f]],
            vals_v.at[f], sems.at[f])
        return 0

    lax.fori_loop(0, N_FIELDS_K - 1, fire, 0)

    pltpu.async_copy(
        wm_hbm.at[0, pl.ds(25 * N_CAT, LAST_LEN)].at[idx_v.at[25]],
        vals_v.at[25], sems.at[25])

    def accum(f, _):
        pltpu.make_async_copy(
            wm_hbm.at[0, pl.ds(0, N_CAT)].at[idx_v.at[0]],
            vals_v.at[f], sems.at[f]).wait()

        def add_chunk(i, _):
            st = i * 16
            acc_v[pl.ds(st, 16)] = acc_v[pl.ds(st, 16)] + vals_v[f, pl.ds(st, 16)]
            return 0

        lax.fori_loop(0, B_PER // 16, add_chunk, 0)
        return 0

    pltpu.make_async_copy(
        wm_hbm.at[0, pl.ds(0, N_CAT)].at[idx_v.at[0]],
        vals_v.at[0], sems.at[0]).wait()
    vbias = aux_v[pl.ds(TAIL, 16)]

    def init_chunk(i, _):
        st = i * 16
        acc_v[pl.ds(st, 16)] = vbias + vals_v[0, pl.ds(st, 16)]
        return 0

    lax.fori_loop(0, B_PER // 16, init_chunk, 0)
    lax.fori_loop(1, N_FIELDS_K - 1, accum, 0)

    pltpu.make_async_copy(
        wm_hbm.at[0, pl.ds(0, N_CAT)].at[idx_v.at[0]],
        vals_v.at[25], sems.at[25]).wait()

    def fix_chunk(i, _):
        st = i * 16
        iv = idx25_v[pl.ds(st, 16)]
        in_tail = iv >= LAST_LEN
        tfix = plsc.load_gather(aux_v, [jnp.minimum(
            jnp.maximum(iv - LAST_LEN, 0), TAIL - 1)])
        v25 = jnp.where(in_tail, tfix, vals_v[25, pl.ds(st, 16)])
        acc_v[pl.ds(st, 16)] = acc_v[pl.ds(st, 16)] + v25
        return 0

    lax.fori_loop(0, B_PER // 16, fix_chunk, 0)

    pltpu.sync_copy(acc_v, out_hbm.at[pl.ds(base, B_PER)])


@jax.jit
def _cat_linear(x_t, w_main, aux):
    mesh = plsc.VectorSubcoreMesh(core_axis_name="c", subcore_axis_name="s")
    f = pl.kernel(
        _body,
        out_type=jax.ShapeDtypeStruct((B_TOTAL,), jnp.float32),
        mesh=mesh,
        compiler_params=pltpu.CompilerParams(
            needs_layout_passes=False,
            skip_device_barrier=True,
            disable_bounds_checks=True,
            disable_semaphore_checks=True,
            use_tc_tiling_on_sc=False,
        ),
        scratch_types=[
            pltpu.VMEM((N_FIELDS_K, B_PER), jnp.int32),
            pltpu.VMEM((N_FIELDS_K, B_PER), jnp.float32),
            pltpu.VMEM((B_PER,), jnp.float32),
            pltpu.VMEM((B_PER,), jnp.int32),
            pltpu.VMEM((TAIL + 16,), jnp.float32),
            pltpu.SemaphoreType.DMA((N_FIELDS_K,)),
        ],
    )
    return f(x_t, w_main, aux)


def kernel(x_cat, W, bias):
    x_t = x_cat.T
    w_row = W.T
    w_main = w_row[:, :SPLIT]
    aux = jnp.concatenate([
        W[SPLIT:, 0],
        jnp.broadcast_to(bias.reshape(()), (16,)).astype(jnp.float32),
    ])
    out = _cat_linear(x_t, w_main, aux)
    return out.reshape(B_TOTAL, 1)
